# baseline (speedup 1.0000x reference)
import functools

import jax
import jax.numpy as jnp
from jax import lax
from jax.experimental import pallas as pl
from jax.experimental.pallas import tpu as pltpu
from jax.experimental.pallas import tpu_sc as plsc

BATCH = 16384
LATENT = 128


def _sc_gather(Eu, Ei, user, items):
    info = plsc.get_sparse_core_info()
    NC, NS = info.num_cores, info.num_subcores
    NW = NC * NS
    bpw = BATCH // NW

    mesh = plsc.VectorSubcoreMesh(core_axis_name="c", subcore_axis_name="s")

    C = bpw // 2

    @functools.partial(
        pl.kernel,
        mesh=mesh,
        out_type=(
            jax.ShapeDtypeStruct((BATCH, LATENT), jnp.float32),
            jax.ShapeDtypeStruct((BATCH, LATENT), jnp.float32),
        ),
        scratch_types=[
            pltpu.VMEM((C,), jnp.int32),
            pltpu.VMEM((C,), jnp.int32),
            pltpu.VMEM((C,), jnp.int32),
            pltpu.VMEM((C,), jnp.int32),
            pltpu.VMEM((C, LATENT), jnp.float32),
            pltpu.VMEM((C, LATENT), jnp.float32),
            pltpu.SemaphoreType.DMA,
            pltpu.SemaphoreType.DMA,
            pltpu.SemaphoreType.DMA,
            pltpu.SemaphoreType.DMA,
        ],
    )
    def k(eu_hbm, ei_hbm, u_hbm, it_hbm, outu_hbm, outi_hbm,
          idxu0, idxu1, idxi0, idxi1, bufa, bufb, gsa, gsb, wsa, wsb):
        wid = lax.axis_index("s") * NC + lax.axis_index("c")
        base = wid * bpw
        pltpu.sync_copy(u_hbm.at[pl.ds(base, C)], idxu0)
        pltpu.sync_copy(u_hbm.at[pl.ds(base + C, C)], idxu1)
        pltpu.sync_copy(it_hbm.at[pl.ds(base, C)], idxi0)
        pltpu.sync_copy(it_hbm.at[pl.ds(base + C, C)], idxi1)
        ga = pltpu.async_copy(eu_hbm.at[idxu0], bufa, gsa)
        gb = pltpu.async_copy(eu_hbm.at[idxu1], bufb, gsb)
        ga.wait()
        wa = pltpu.async_copy(bufa, outu_hbm.at[pl.ds(base, C)], wsa)
        gb.wait()
        wb = pltpu.async_copy(bufb, outu_hbm.at[pl.ds(base + C, C)], wsb)
        wa.wait()
        ga = pltpu.async_copy(ei_hbm.at[idxi0], bufa, gsa)
        wb.wait()
        gb = pltpu.async_copy(ei_hbm.at[idxi1], bufb, gsb)
        ga.wait()
        wa = pltpu.async_copy(bufa, outi_hbm.at[pl.ds(base, C)], wsa)
        gb.wait()
        wb = pltpu.async_copy(bufb, outi_hbm.at[pl.ds(base + C, C)], wsb)
        wa.wait()
        wb.wait()

    return k(Eu, Ei, user, items)


def _mlp_body(ue_ref, ie_ref, w1u_ref, w1i_ref, b1_ref, w2_ref, b2_ref,
              w3_ref, b3_ref, w4_ref, b4_ref, out_ref):
    x = jnp.dot(ue_ref[...], w1u_ref[...], preferred_element_type=jnp.float32)
    x = x + jnp.dot(ie_ref[...], w1i_ref[...], preferred_element_type=jnp.float32)
    x = jnp.maximum(x + b1_ref[...], 0.0)
    x = jnp.maximum(jnp.dot(x, w2_ref[...], preferred_element_type=jnp.float32) + b2_ref[...], 0.0)
    x = jnp.maximum(jnp.dot(x, w3_ref[...], preferred_element_type=jnp.float32) + b3_ref[...], 0.0)
    x = jnp.dot(x, w4_ref[...], preferred_element_type=jnp.float32) + b4_ref[...]
    out_ref[...] = 1.0 / (1.0 + jnp.exp(-x))


def _tc_mlp(ue, ie, W1, b1, W2, b2, W3, b3, W4, b4):
    BLK = 2048
    grid = (BATCH // BLK,)
    w1u = W1[:, :LATENT].T
    w1i = W1[:, LATENT:].T
    w2t = W2.T
    w3t = W3.T
    w4t = W4.T
    b1r = b1.reshape(1, -1)
    b2r = b2.reshape(1, -1)
    b3r = b3.reshape(1, -1)
    b4r = b4.reshape(1, -1)

    def full(shape):
        return pl.BlockSpec(shape, lambda i: (0, 0))

    return pl.pallas_call(
        _mlp_body,
        grid=grid,
        in_specs=[
            pl.BlockSpec((BLK, LATENT), lambda i: (i, 0)),
            pl.BlockSpec((BLK, LATENT), lambda i: (i, 0)),
            full(w1u.shape), full(w1i.shape), full(b1r.shape),
            full(w2t.shape), full(b2r.shape),
            full(w3t.shape), full(b3r.shape),
            full(w4t.shape), full(b4r.shape),
        ],
        out_specs=pl.BlockSpec((BLK, 1), lambda i: (i, 0)),
        out_shape=jax.ShapeDtypeStruct((BATCH, 1), jnp.float32),
    )(ue, ie, w1u, w1i, b1r, w2t, b2r, w3t, b3r, w4t, b4r)


def kernel(user, items, Eu, Ei, W1, b1, W2, b2, W3, b3, W4, b4):
    ue, ie = _sc_gather(Eu, Ei, user.astype(jnp.int32), items.astype(jnp.int32))
    return _tc_mlp(ue, ie, W1, b1, W2, b2, W3, b3, W4, b4)


# speedup vs baseline: 1.0313x; 1.0313x over previous
import functools

import jax
import jax.numpy as jnp
from jax import lax
from jax.experimental import pallas as pl
from jax.experimental.pallas import tpu as pltpu
from jax.experimental.pallas import tpu_sc as plsc

BATCH = 16384
LATENT = 128


def _sc_gather(Eu, Ei, user, items, batch):
    info = plsc.get_sparse_core_info()
    NC, NS = info.num_cores, info.num_subcores
    NW = NC * NS
    bpw = batch // NW

    mesh = plsc.VectorSubcoreMesh(core_axis_name="c", subcore_axis_name="s")

    @functools.partial(
        pl.kernel,
        mesh=mesh,
        out_type=(
            jax.ShapeDtypeStruct((batch, LATENT), jnp.float32),
            jax.ShapeDtypeStruct((batch, LATENT), jnp.float32),
        ),
        scratch_types=[
            pltpu.VMEM((bpw,), jnp.int32),
            pltpu.VMEM((bpw, LATENT), jnp.float32),
            pltpu.SemaphoreType.DMA,
        ],
    )
    def k(eu_hbm, ei_hbm, u_hbm, it_hbm, outu_hbm, outi_hbm, idx_v, rows_v, sem):
        wid = lax.axis_index("s") * NC + lax.axis_index("c")
        base = wid * bpw
        pltpu.sync_copy(u_hbm.at[pl.ds(base, bpw)], idx_v)
        pltpu.async_copy(eu_hbm.at[idx_v], rows_v, sem).wait()
        pltpu.sync_copy(rows_v, outu_hbm.at[pl.ds(base, bpw)])
        pltpu.sync_copy(it_hbm.at[pl.ds(base, bpw)], idx_v)
        pltpu.async_copy(ei_hbm.at[idx_v], rows_v, sem).wait()
        pltpu.sync_copy(rows_v, outi_hbm.at[pl.ds(base, bpw)])

    return k(Eu, Ei, user, items)


def _mlp_body(ue_ref, ie_ref, w1u_ref, w1i_ref, b1_ref, w2_ref, b2_ref,
              w3_ref, b3_ref, w4_ref, b4_ref, out_ref):
    x = jnp.dot(ue_ref[...], w1u_ref[...], preferred_element_type=jnp.float32)
    x = x + jnp.dot(ie_ref[...], w1i_ref[...], preferred_element_type=jnp.float32)
    x = jnp.maximum(x + b1_ref[...], 0.0)
    x = jnp.maximum(jnp.dot(x, w2_ref[...], preferred_element_type=jnp.float32) + b2_ref[...], 0.0)
    x = jnp.maximum(jnp.dot(x, w3_ref[...], preferred_element_type=jnp.float32) + b3_ref[...], 0.0)
    x = jnp.dot(x, w4_ref[...], preferred_element_type=jnp.float32) + b4_ref[...]
    out_ref[...] = 1.0 / (1.0 + jnp.exp(-x))


def _tc_mlp(ue, ie, w1u, w1i, b1r, w2t, b2r, w3t, b3r, w4t, b4r):
    batch = ue.shape[0]
    BLK = 2048
    grid = (batch // BLK,)

    def full(shape):
        return pl.BlockSpec(shape, lambda i: (0, 0))

    return pl.pallas_call(
        _mlp_body,
        grid=grid,
        in_specs=[
            pl.BlockSpec((BLK, LATENT), lambda i: (i, 0)),
            pl.BlockSpec((BLK, LATENT), lambda i: (i, 0)),
            full(w1u.shape), full(w1i.shape), full(b1r.shape),
            full(w2t.shape), full(b2r.shape),
            full(w3t.shape), full(b3r.shape),
            full(w4t.shape), full(b4r.shape),
        ],
        out_specs=pl.BlockSpec((BLK, 1), lambda i: (i, 0)),
        out_shape=jax.ShapeDtypeStruct((batch, 1), jnp.float32),
    )(ue, ie, w1u, w1i, b1r, w2t, b2r, w3t, b3r, w4t, b4r)


def kernel(user, items, Eu, Ei, W1, b1, W2, b2, W3, b3, W4, b4):
    u32 = user.astype(jnp.int32)
    i32 = items.astype(jnp.int32)
    w1u = W1[:, :LATENT].T
    w1i = W1[:, LATENT:].T
    w2t = W2.T
    w3t = W3.T
    w4t = W4.T
    b1r = b1.reshape(1, -1)
    b2r = b2.reshape(1, -1)
    b3r = b3.reshape(1, -1)
    b4r = b4.reshape(1, -1)
    H = BATCH // 2
    ue0, ie0 = _sc_gather(Eu, Ei, u32[:H], i32[:H], H)
    ue1, ie1 = _sc_gather(Eu, Ei, u32[H:], i32[H:], H)
    y0 = _tc_mlp(ue0, ie0, w1u, w1i, b1r, w2t, b2r, w3t, b3r, w4t, b4r)
    y1 = _tc_mlp(ue1, ie1, w1u, w1i, b1r, w2t, b2r, w3t, b3r, w4t, b4r)
    return jnp.concatenate([y0, y1], axis=0)
